# Initial kernel scaffold; baseline (speedup 1.0000x reference)
import jax
import jax.numpy as jnp
from jax.experimental import pallas as pl


def kernel(x, table):
    raise NotImplementedError("write your pallas kernel here")


# speedup vs baseline: 1.9692x; 1.9692x over previous
import functools

import jax
import jax.numpy as jnp
from jax import lax
from jax.experimental import pallas as pl
from jax.experimental.pallas import tpu as pltpu
from jax.experimental.pallas import tpu_sc as plsc

D_MODEL = 1024
NUM_CORES = 2
NUM_SUBCORES = 16
NW = NUM_CORES * NUM_SUBCORES
CHUNK = 32


@functools.cache
def _make_lookup(B):
    b_per_w = B // NW
    nchunk = b_per_w // CHUNK
    mesh = plsc.VectorSubcoreMesh(core_axis_name="c", subcore_axis_name="s")

    @functools.partial(
        pl.kernel,
        mesh=mesh,
        out_type=jax.ShapeDtypeStruct((B, D_MODEL), jnp.float32),
        scratch_types=[
            pltpu.VMEM((nchunk, CHUNK), jnp.int32),
            pltpu.VMEM((2, CHUNK, D_MODEL), jnp.float32),
            pltpu.SemaphoreType.DMA,
        ],
    )
    def lookup(idx_hbm, table_hbm, out_hbm, idx_v, rows_v, gsem):
        wid = lax.axis_index("s") * NUM_CORES + lax.axis_index("c")
        base = wid * b_per_w
        pltpu.sync_copy(idx_hbm.at[wid], idx_v)
        gathers = [None] * nchunk
        gathers[0] = pltpu.async_copy(
            table_hbm.at[idx_v.at[0]], rows_v.at[0], gsem)
        for j in range(nchunk):
            gathers[j].wait()
            if j + 1 < nchunk:
                gathers[j + 1] = pltpu.async_copy(
                    table_hbm.at[idx_v.at[j + 1]], rows_v.at[(j + 1) % 2],
                    gsem)
            pltpu.sync_copy(rows_v.at[j % 2],
                            out_hbm.at[pl.ds(base + j * CHUNK, CHUNK)])

    return lookup


def kernel(x, table):
    B = x.size
    idx = jnp.reshape(x.astype(jnp.int32), (NW, B // NW // CHUNK, CHUNK))
    out = _make_lookup(B)(idx, table)
    return jnp.reshape(out, x.shape + (D_MODEL,))


# speedup vs baseline: 2.0732x; 1.0528x over previous
import functools

import jax
import jax.numpy as jnp
from jax import lax
from jax.experimental import pallas as pl
from jax.experimental.pallas import tpu as pltpu
from jax.experimental.pallas import tpu_sc as plsc

D_MODEL = 1024
NUM_CORES = 2
NUM_SUBCORES = 16
NW = NUM_CORES * NUM_SUBCORES
CHUNK = 32


@functools.cache
def _make_lookup(B):
    b_per_w = B // NW
    nchunk = b_per_w // CHUNK
    mesh = plsc.VectorSubcoreMesh(core_axis_name="c", subcore_axis_name="s")

    nbuf = 3

    @functools.partial(
        pl.kernel,
        mesh=mesh,
        out_type=jax.ShapeDtypeStruct((B, D_MODEL), jnp.float32),
        scratch_types=[
            pltpu.VMEM((nchunk, CHUNK), jnp.int32),
            pltpu.VMEM((nbuf, CHUNK, D_MODEL), jnp.float32),
            pltpu.SemaphoreType.DMA,
            pltpu.SemaphoreType.DMA,
        ],
    )
    def lookup(idx_hbm, table_hbm, out_hbm, idx_v, rows_v, gsem, ssem):
        wid = lax.axis_index("s") * NUM_CORES + lax.axis_index("c")
        base = wid * b_per_w
        pltpu.sync_copy(idx_hbm.at[wid], idx_v)
        gathers = [None] * nchunk
        stores = [None] * nchunk
        for b in range(min(nbuf, nchunk)):
            gathers[b] = pltpu.async_copy(
                table_hbm.at[idx_v.at[b]], rows_v.at[b], gsem)
        for j in range(nchunk):
            gathers[j].wait()
            stores[j] = pltpu.async_copy(
                rows_v.at[j % nbuf],
                out_hbm.at[pl.ds(base + j * CHUNK, CHUNK)], ssem)
            g = j + nbuf - 1
            if j >= 1 and g < nchunk:
                stores[j - 1].wait()
                gathers[g] = pltpu.async_copy(
                    table_hbm.at[idx_v.at[g]], rows_v.at[g % nbuf], gsem)
        for j in range(max(0, nchunk - nbuf), nchunk):
            stores[j].wait()

    return lookup


def kernel(x, table):
    B = x.size
    idx = jnp.reshape(x.astype(jnp.int32), (NW, B // NW // CHUNK, CHUNK))
    out = _make_lookup(B)(idx, table)
    return jnp.reshape(out, x.shape + (D_MODEL,))
